# baseline (speedup 1.0000x reference)
import functools

import jax
import jax.numpy as jnp
from jax import lax
from jax.experimental import pallas as pl
from jax.experimental.pallas import tpu as pltpu
from jax.experimental.pallas import tpu_sc as plsc

_B = 16384
_C = 784
_NB = 64
_NC = 208
_NN = 512
_KS = 16
_L = 16
_NCORES = 2
_NSUB = 16
_NW = _NCORES * _NSUB
_ROWS_PER_W = _B // _NW
_GROUPS = _ROWS_PER_W // _L
_VPC = _C // _L


def _sc_partial_rowsum(x2d, a, cvec):
    mesh = plsc.VectorSubcoreMesh(core_axis_name="c", subcore_axis_name="s")

    @functools.partial(
        pl.kernel,
        mesh=mesh,
        out_type=jax.ShapeDtypeStruct((_B, _L), jnp.float32),
        scratch_types=[
            pltpu.VMEM((_L, _C), jnp.float32),
            pltpu.VMEM((_L, _L), jnp.float32),
            pltpu.VMEM((_C,), jnp.float32),
            pltpu.VMEM((_L,), jnp.float32),
        ],
    )
    def k(x_hbm, a_hbm, c_hbm, p_hbm, buf, gbuf, a_v, c_v):
        wid = lax.axis_index("s") * _NCORES + lax.axis_index("c")
        base = wid * _ROWS_PER_W
        pltpu.sync_copy(a_hbm, a_v)
        pltpu.sync_copy(c_hbm, c_v)
        cv = c_v[...]

        def group(g, carry):
            row0 = base + g * _L
            pltpu.sync_copy(x_hbm.at[pl.ds(row0, _L)], buf)
            for rr in range(_L):
                acc = cv
                for j in range(_VPC):
                    acc = acc + buf[rr, pl.ds(j * _L, _L)] * a_v[pl.ds(j * _L, _L)]
                gbuf[rr, :] = acc
            pltpu.sync_copy(gbuf, p_hbm.at[pl.ds(row0, _L)])
            return carry

        lax.fori_loop(0, _GROUPS, group, 0)

    return k(x2d, a, cvec)


def _tc_final_rowsum(p):
    def body(p_ref, o_ref):
        o_ref[...] = jnp.sum(p_ref[...], axis=1, keepdims=True)

    return pl.pallas_call(
        body,
        out_shape=jax.ShapeDtypeStruct((_B, 1), jnp.float32),
        grid=(8,),
        in_specs=[pl.BlockSpec((_B // 8, _L), lambda i: (i, 0))],
        out_specs=pl.BlockSpec((_B // 8, 1), lambda i: (i, 0)),
    )(p)


def kernel(input_linear, w1, b1, w2, b2, K, train_size, num_cat_variable,
           num_num_variable, num_bin_variable,
           vectorized_cate_col_name_num_list):
    x2d = input_linear.reshape(_B, _C)

    counts = jnp.asarray(vectorized_cate_col_name_num_list, dtype=jnp.int32)
    cum = jnp.cumsum(counts)
    seg = jnp.searchsorted(cum, jnp.arange(_NC, dtype=jnp.int32), side="right")
    seg = jnp.minimum(seg.astype(jnp.int32), _NC - 1)
    gmap = jnp.concatenate([
        jnp.arange(_NB, dtype=jnp.int32),
        _NB + seg,
        _NB + _NC + jnp.arange(_NN, dtype=jnp.int32) // _KS,
    ])
    w2g = w2[gmap]
    a = w1 * w2g
    cconst = jnp.dot(b1, w2g) + jnp.sum(b2)
    cvec = jnp.full((_L,), cconst / _L, dtype=jnp.float32)

    p = _sc_partial_rowsum(x2d, a, cvec)
    return _tc_final_rowsum(p)


# speedup vs baseline: 1.7164x; 1.7164x over previous
import functools

import jax
import jax.numpy as jnp
from jax import lax
from jax.experimental import pallas as pl
from jax.experimental.pallas import tpu as pltpu
from jax.experimental.pallas import tpu_sc as plsc

_B = 16384
_C = 784
_NB = 64
_NC = 208
_NN = 512
_KS = 16
_L = 16
_NCORES = 2
_NSUB = 16
_NW = _NCORES * _NSUB
_ROWS_PER_W = _B // _NW
_GROUPS = _ROWS_PER_W // _L
_VPC = _C // _L


def _sc_partial_rowsum(x2d, a, cvec):
    mesh = plsc.VectorSubcoreMesh(core_axis_name="c", subcore_axis_name="s")

    @functools.partial(
        pl.kernel,
        mesh=mesh,
        out_type=jax.ShapeDtypeStruct((_B, _L), jnp.float32),
        scratch_types=[
            pltpu.VMEM((_L, _C), jnp.float32),
            pltpu.VMEM((_L, _C), jnp.float32),
            pltpu.VMEM((_L, _L), jnp.float32),
            pltpu.VMEM((_L, _L), jnp.float32),
            pltpu.VMEM((_C,), jnp.float32),
            pltpu.VMEM((_L,), jnp.float32),
            pltpu.SemaphoreType.DMA,
            pltpu.SemaphoreType.DMA,
            pltpu.SemaphoreType.DMA,
            pltpu.SemaphoreType.DMA,
        ],
    )
    def k(x_hbm, a_hbm, c_hbm, p_hbm, buf0, buf1, pb0, pb1, a_v, c_v,
          isem0, isem1, osem0, osem1):
        wid = lax.axis_index("s") * _NCORES + lax.axis_index("c")
        base = wid * _ROWS_PER_W
        pltpu.sync_copy(a_hbm, a_v)
        pltpu.sync_copy(c_hbm, c_v)
        cv = c_v[...]

        bufs = (buf0, buf1)
        pbs = (pb0, pb1)
        isems = (isem0, isem1)
        osems = (osem0, osem1)

        def in_slice(g):
            return x_hbm.at[pl.ds(base + g * _L, _L)]

        def out_slice(g):
            return p_hbm.at[pl.ds(base + g * _L, _L)]

        pltpu.async_copy(in_slice(0), buf0, isem0)

        def step(i, carry):
            for s in range(2):
                g = 2 * i + s
                buf, pb = bufs[s], pbs[s]
                isem, osem = isems[s], osems[s]
                o = 1 - s
                gn = g + 1

                @pl.when(gn < _GROUPS)
                def _():
                    pltpu.async_copy(in_slice(gn), bufs[o], isems[o])

                pltpu.make_async_copy(in_slice(g), buf, isem).wait()
                @pl.when(i > 0)
                def _():
                    pltpu.make_async_copy(pb, out_slice(g), osem).wait()
                accs = [cv] * _L
                for j in range(_VPC):
                    aj = a_v[pl.ds(j * _L, _L)]
                    for rr in range(_L):
                        accs[rr] = accs[rr] + buf[rr, pl.ds(j * _L, _L)] * aj
                for rr in range(_L):
                    pb[rr, :] = accs[rr]
                pltpu.async_copy(pb, out_slice(g), osem)
            return carry

        lax.fori_loop(0, _GROUPS // 2, step, 0)
        pltpu.make_async_copy(pb0, out_slice(_GROUPS - 2), osem0).wait()
        pltpu.make_async_copy(pb1, out_slice(_GROUPS - 1), osem1).wait()

    return k(x2d, a, cvec)


def _tc_final_rowsum(p):
    def body(p_ref, o_ref):
        o_ref[...] = jnp.sum(p_ref[...], axis=1, keepdims=True)

    return pl.pallas_call(
        body,
        out_shape=jax.ShapeDtypeStruct((_B, 1), jnp.float32),
        grid=(8,),
        in_specs=[pl.BlockSpec((_B // 8, _L), lambda i: (i, 0))],
        out_specs=pl.BlockSpec((_B // 8, 1), lambda i: (i, 0)),
    )(p)


def kernel(input_linear, w1, b1, w2, b2, K, train_size, num_cat_variable,
           num_num_variable, num_bin_variable,
           vectorized_cate_col_name_num_list):
    x2d = input_linear.reshape(_B, _C)

    counts = jnp.asarray(vectorized_cate_col_name_num_list, dtype=jnp.int32)
    cum = jnp.cumsum(counts)
    seg = jnp.searchsorted(cum, jnp.arange(_NC, dtype=jnp.int32), side="right")
    seg = jnp.minimum(seg.astype(jnp.int32), _NC - 1)
    gmap = jnp.concatenate([
        jnp.arange(_NB, dtype=jnp.int32),
        _NB + seg,
        _NB + _NC + jnp.arange(_NN, dtype=jnp.int32) // _KS,
    ])
    w2g = w2[gmap]
    a = w1 * w2g
    cconst = jnp.dot(b1, w2g) + jnp.sum(b2)
    cvec = jnp.full((_L,), cconst / _L, dtype=jnp.float32)

    p = _sc_partial_rowsum(x2d, a, cvec)
    return _tc_final_rowsum(p)
